# baseline (speedup 1.0000x reference)
import functools

import jax
import jax.numpy as jnp
from jax import lax
from jax.experimental import pallas as pl
from jax.experimental.pallas import tpu as pltpu
from jax.experimental.pallas import tpu_sc as plsc

_VOCAB = 1000000
_EMBED_DIM = 64
_BATCH = 16384

_INFO = plsc.get_sparse_core_info()
_NC, _NS, _L = _INFO.num_cores, _INFO.num_subcores, _INFO.num_lanes
_NW = _NC * _NS
_BPW = _BATCH // _NW
_STRIPE = _VOCAB // _NW
_NIDX = 2 * _BATCH
_ICH = 2048
_QCAP = 2048
_GRP = 32
_PW = 512
_MAXOFF = 1000064 - _PW
_SROWS = _NIDX + 1


def _body_a(tgt_hbm, ctx_hbm, tab_t_hbm, stage_hbm,
            ichunk, hits, q_v, q_m, pan0, pan1, bounce,
            psem0, psem1, wsem):
    wid = lax.axis_index("s") * _NC + lax.axis_index("c")
    lo = wid * _STRIPE
    hi = lo + _STRIPE
    lane = lax.iota(jnp.int32, _L)

    def scan_table(nhit, idx_hbm, mbase):
        def chunk(j, nh):
            pltpu.sync_copy(idx_hbm.at[pl.ds(j * _ICH, _ICH)], ichunk)

            def vec(k, nh2):
                v = ichunk[pl.ds(k * _L, _L)]
                m = (v >= lo) & (v < hi)
                pos = mbase + j * _ICH + k * _L + lane
                packed = ((v - lo) << 15) | pos
                plsc.store_compressed(hits.at[pl.ds(nh2, _L)], packed, mask=m)
                return nh2 + plsc.all_reduce_population_count(m)[0]

            return lax.fori_loop(0, _ICH // _L, vec, nh)

        return lax.fori_loop(0, _BATCH // _ICH, chunk, nhit)

    nhit = scan_table(jnp.int32(0), tgt_hbm, 0)
    nhit = scan_table(nhit, ctx_hbm, _BATCH)
    nh16 = (nhit + _L - 1) // _L

    p0 = lo // _PW
    p1 = (hi + _PW - 1) // _PW
    npan = p1 - p0
    npairs = (npan + 1) // 2

    def off_of(p):
        return jnp.minimum(p * _PW, _MAXOFF)

    def fetch(p, pan, psem):
        pltpu.make_async_copy(
            tab_t_hbm.at[:, pl.ds(pl.multiple_of(off_of(p), 128), _PW)],
            pan, psem).start()

    def drain_panel(pan, psem):
        pltpu.make_async_copy(
            tab_t_hbm.at[:, pl.ds(0, _PW)], pan, psem).wait()

    def process(p, pan):
        off = off_of(p)
        plo = p * _PW
        phi = plo + _PW

        def find(h, qc):
            hv = (hits[pl.ds(h * _L, _L)] >> 15) + lo
            hm = hits[pl.ds(h * _L, _L)] & 0x7FFF
            valid = (h * _L + lane) < nhit
            m = valid & (hv >= plo) & (hv < phi)
            qcc = jnp.minimum(qc, _QCAP - _L)
            plsc.store_compressed(q_v.at[pl.ds(qcc, _L)], hv - off, mask=m)
            plsc.store_compressed(q_m.at[pl.ds(qcc, _L)], hm, mask=m)
            return qcc + plsc.all_reduce_population_count(m)[0]

        qc = lax.fori_loop(0, nh16, find, jnp.int32(0))
        alltrue = lane < _L
        for pad in range(_GRP // _L):
            plsc.store_compressed(q_v.at[pl.ds(qc + pad * _L, _L)],
                                  jnp.zeros((_L,), jnp.int32), mask=alltrue)
            plsc.store_compressed(q_m.at[pl.ds(qc + pad * _L, _L)],
                                  jnp.full((_L,), _NIDX, jnp.int32),
                                  mask=alltrue)

        def group(e, carry2):
            base = e * _GRP
            for shalf in range(_GRP // _L):
                qv = q_v[pl.ds(base + shalf * _L, _L)]
                qm = q_m[pl.ds(base + shalf * _L, _L)]
                for s in range(_L):
                    slot = shalf * _L + s
                    col = qv[s]
                    m0 = qm[s]
                    for k in range(4):
                        g = plsc.load_gather(
                            pan, [lane + _L * k,
                                  jnp.full((_L,), col, jnp.int32)])
                        bounce[pl.ds(slot * _EMBED_DIM + _L * k, _L)] = g
                    pltpu.make_async_copy(
                        bounce.at[pl.ds(slot * _EMBED_DIM, _EMBED_DIM)],
                        stage_hbm.at[pl.ds(m0 * _EMBED_DIM, _EMBED_DIM)],
                        wsem).start()
            pltpu.make_async_copy(
                stage_hbm.at[pl.ds(0, _GRP * _EMBED_DIM)], bounce, wsem).wait()
            return carry2

        lax.fori_loop(0, (qc + _GRP - 1) // _GRP, group, 0)

    fetch(p0, pan0, psem0)

    def pair(k, carry):
        pa = p0 + 2 * k
        fetch(pa + 1, pan1, psem1)
        drain_panel(pan0, psem0)
        process(pa, pan0)
        fetch(pa + 2, pan0, psem0)
        drain_panel(pan1, psem1)
        process(pa + 1, pan1)
        return carry

    lax.fori_loop(0, npairs, pair, 0)
    drain_panel(pan0, psem0)


def _body_b(stage_hbm, out_hbm, rows_t, rows_c, out_v):
    wid = lax.axis_index("s") * _NC + lax.axis_index("c")
    lane = lax.iota(jnp.int32, _L)

    pltpu.sync_copy(
        stage_hbm.at[pl.ds(wid * _BPW * _EMBED_DIM, _BPW * _EMBED_DIM)],
        rows_t)
    pltpu.sync_copy(
        stage_hbm.at[pl.ds((_BATCH + wid * _BPW) * _EMBED_DIM,
                           _BPW * _EMBED_DIM)],
        rows_c)

    def body(g, carry):
        elem0 = (g * _L + lane) * _EMBED_DIM
        acc = jnp.zeros((_L,), jnp.float32)
        for d in range(_EMBED_DIM):
            t = plsc.load_gather(rows_t, [elem0 + d])
            c = plsc.load_gather(rows_c, [elem0 + d])
            acc = acc + t * c
        e = jnp.exp(-jnp.abs(acc))
        r_ = 1.0 / (1.0 + e)
        sig = jnp.where(acc >= 0, r_, e * r_)
        out_v[pl.ds(g * _L, _L)] = sig
        return carry

    lax.fori_loop(0, _BPW // _L, body, 0)

    pltpu.sync_copy(out_v, out_hbm.at[pl.ds(wid * _BPW, _BPW)])


@jax.jit
def _run(target_i, context_j, shared_embedding):
    mesh = plsc.VectorSubcoreMesh(core_axis_name="c", subcore_axis_name="s")
    kern_a = functools.partial(
        pl.kernel,
        out_type=jax.ShapeDtypeStruct((_SROWS * _EMBED_DIM,), jnp.float32),
        mesh=mesh,
        scratch_types=[
            pltpu.VMEM((_ICH,), jnp.int32),
            pltpu.VMEM((_NIDX,), jnp.int32),
            pltpu.VMEM((_QCAP + _GRP,), jnp.int32),
            pltpu.VMEM((_QCAP + _GRP,), jnp.int32),
            pltpu.VMEM((_EMBED_DIM, _PW), jnp.float32),
            pltpu.VMEM((_EMBED_DIM, _PW), jnp.float32),
            pltpu.VMEM((_GRP * _EMBED_DIM,), jnp.float32),
            pltpu.SemaphoreType.DMA,
            pltpu.SemaphoreType.DMA,
            pltpu.SemaphoreType.DMA,
        ],
        compiler_params=pltpu.CompilerParams(needs_layout_passes=False),
    )(_body_a)
    kern_b = functools.partial(
        pl.kernel,
        out_type=jax.ShapeDtypeStruct((_BATCH,), jnp.float32),
        mesh=mesh,
        scratch_types=[
            pltpu.VMEM((_BPW * _EMBED_DIM,), jnp.float32),
            pltpu.VMEM((_BPW * _EMBED_DIM,), jnp.float32),
            pltpu.VMEM((_BPW,), jnp.float32),
        ],
        compiler_params=pltpu.CompilerParams(needs_layout_passes=False),
    )(_body_b)
    stage = kern_a(target_i, context_j, shared_embedding.T)
    return kern_b(stage)


def kernel(target_i, context_j, shared_embedding):
    return _run(target_i.astype(jnp.int32), context_j.astype(jnp.int32),
                shared_embedding)


# speedup vs baseline: 1.5388x; 1.5388x over previous
import functools

import jax
import jax.numpy as jnp
from jax import lax
from jax.experimental import pallas as pl
from jax.experimental.pallas import tpu as pltpu
from jax.experimental.pallas import tpu_sc as plsc

_VOCAB = 1000000
_EMBED_DIM = 64
_BATCH = 16384

_INFO = plsc.get_sparse_core_info()
_NC, _NS, _L = _INFO.num_cores, _INFO.num_subcores, _INFO.num_lanes
_NW = _NC * _NS
_BPW = _BATCH // _NW
_CHUNK = 128
_NCHUNK = _BPW // _CHUNK


def _sc_body(tgt_hbm, ctx_hbm, table_hbm, out_hbm,
             idx_t, idx_c, rows_t, rows_c, out_v, sem):
    wid = lax.axis_index("s") * _NC + lax.axis_index("c")
    row0 = wid * _NCHUNK

    pltpu.sync_copy(tgt_hbm.at[pl.ds(row0, _NCHUNK)], idx_t)
    pltpu.sync_copy(ctx_hbm.at[pl.ds(row0, _NCHUNK)], idx_c)

    copies = []
    for j in range(_NCHUNK):
        copies.append(pltpu.async_copy(
            table_hbm.at[idx_t.at[j]],
            rows_t.at[pl.ds(j * _CHUNK, _CHUNK)], sem))
        copies.append(pltpu.async_copy(
            table_hbm.at[idx_c.at[j]],
            rows_c.at[pl.ds(j * _CHUNK, _CHUNK)], sem))
    for c in copies:
        c.wait()

    lane = lax.iota(jnp.int32, _L)

    def body(g, carry):
        row_ids = g * _L + lane
        acc = jnp.zeros((_L,), jnp.float32)
        for d in range(_EMBED_DIM):
            dim_ids = jnp.full((_L,), d, jnp.int32)
            t = plsc.load_gather(rows_t, [row_ids, dim_ids])
            c = plsc.load_gather(rows_c, [row_ids, dim_ids])
            acc = acc + t * c
        e = jnp.exp(-jnp.abs(acc))
        r = 1.0 / (1.0 + e)
        sig = jnp.where(acc >= 0, r, e * r)
        out_v[pl.ds(g * _L, _L)] = sig
        return carry

    lax.fori_loop(0, _BPW // _L, body, 0)

    pltpu.sync_copy(out_v, out_hbm.at[pl.ds(wid * _BPW, _BPW)])


@jax.jit
def _run(target_i, context_j, shared_embedding):
    mesh = plsc.VectorSubcoreMesh(core_axis_name="c", subcore_axis_name="s")
    tgt2d = target_i.reshape(_NW * _NCHUNK, _CHUNK)
    ctx2d = context_j.reshape(_NW * _NCHUNK, _CHUNK)
    kern = functools.partial(
        pl.kernel,
        out_type=jax.ShapeDtypeStruct((_BATCH,), jnp.float32),
        mesh=mesh,
        scratch_types=[
            pltpu.VMEM((_NCHUNK, _CHUNK), jnp.int32),
            pltpu.VMEM((_NCHUNK, _CHUNK), jnp.int32),
            pltpu.VMEM((_BPW, _EMBED_DIM), jnp.float32),
            pltpu.VMEM((_BPW, _EMBED_DIM), jnp.float32),
            pltpu.VMEM((_BPW,), jnp.float32),
            pltpu.SemaphoreType.DMA,
        ],
        compiler_params=pltpu.CompilerParams(
            needs_layout_passes=False, use_tc_tiling_on_sc=False),
    )(_sc_body)
    return kern(tgt2d, ctx2d, shared_embedding)


def kernel(target_i, context_j, shared_embedding):
    return _run(target_i.astype(jnp.int32), context_j.astype(jnp.int32),
                shared_embedding)
